# baseline (speedup 1.0000x reference)
import functools

import jax
import jax.numpy as jnp
from jax import lax
from jax.experimental import pallas as pl
from jax.experimental.pallas import tpu as pltpu
from jax.experimental.pallas import tpu_sc as plsc

VOCAB_ROWS = 1000000
EMBED_DIM = 64
PAD_DIM = 128
TOTAL = 4096 * 200

_info = plsc.get_sparse_core_info()
NUM_CORES = _info.num_cores
NUM_SUBCORES = _info.num_subcores
NW = NUM_CORES * NUM_SUBCORES
PER_W = TOTAL // NW
CHUNK = 800
NCHUNK = PER_W // CHUNK


def _gather_body(table_hbm, idx_hbm, out_hbm, idx_v,
                 rows0, rows1, gsem0, gsem1, ssem0, ssem1):
    wid = lax.axis_index("s") * NUM_CORES + lax.axis_index("c")
    base = wid * PER_W
    pltpu.sync_copy(idx_hbm.at[pl.ds(base, PER_W)], idx_v)

    rows = (rows0, rows1)
    gsem = (gsem0, gsem1)
    ssem = (ssem0, ssem1)

    def g_desc(j, b):
        return pltpu.make_async_copy(
            table_hbm.at[idx_v.at[pl.ds(j * CHUNK, CHUNK)]], rows[b], gsem[b])

    def s_desc(j, b):
        return pltpu.make_async_copy(
            rows[b], out_hbm.at[pl.ds(base + j * CHUNK, CHUNK)], ssem[b])

    g_desc(0, 0).start()
    g_desc(1, 1).start()

    def step(j0, carry):
        for b in range(2):
            j = j0 * 2 + b
            g_desc(j, b).wait()
            s_desc(j, b).start()

            @pl.when(j + 2 < NCHUNK)
            def _():
                s_desc(j, b).wait()
                g_desc(j + 2, b).start()

        return carry

    lax.fori_loop(0, NCHUNK // 2, step, 0, unroll=False)

    s_desc(NCHUNK - 2, 0).wait()
    s_desc(NCHUNK - 1, 1).wait()


@jax.jit
def _run(indices, table):
    table2 = jnp.pad(table, ((0, 0), (0, PAD_DIM - EMBED_DIM)))
    table2 = table2.reshape(2 * VOCAB_ROWS, EMBED_DIM)
    idx2 = indices.reshape(-1) * 2
    mesh = plsc.VectorSubcoreMesh(core_axis_name="c", subcore_axis_name="s")
    grab = pl.kernel(
        _gather_body,
        out_type=jax.ShapeDtypeStruct((TOTAL, EMBED_DIM), jnp.float32),
        mesh=mesh,
        scratch_types=[
            pltpu.VMEM((PER_W,), jnp.int32),
            pltpu.VMEM((CHUNK, EMBED_DIM), jnp.float32),
            pltpu.VMEM((CHUNK, EMBED_DIM), jnp.float32),
            pltpu.SemaphoreType.DMA,
            pltpu.SemaphoreType.DMA,
            pltpu.SemaphoreType.DMA,
            pltpu.SemaphoreType.DMA,
        ],
        compiler_params=pltpu.CompilerParams(use_tc_tiling_on_sc=False),
    )
    out = grab(table2, idx2)
    return out.reshape(indices.shape[0], indices.shape[1], EMBED_DIM)


def kernel(indices, embedding_table):
    return _run(indices, embedding_table)


# speedup vs baseline: 1.3558x; 1.3558x over previous
import functools

import jax
import jax.numpy as jnp
from jax import lax
from jax.experimental import pallas as pl
from jax.experimental.pallas import tpu as pltpu
from jax.experimental.pallas import tpu_sc as plsc

VOCAB_ROWS = 1000000
EMBED_DIM = 64
PAD_DIM = 128
TOTAL = 4096 * 200

_info = plsc.get_sparse_core_info()
NUM_CORES = _info.num_cores
NUM_SUBCORES = _info.num_subcores
NW = NUM_CORES * NUM_SUBCORES
PER_W = TOTAL // NW
CHUNK = 800
NCHUNK = PER_W // CHUNK


def _gather_body(table_hbm, idx_hbm, out_hbm, idx_v,
                 rows0, rows1, gsem0, gsem1, ssem0, ssem1):
    wid = lax.axis_index("s") * NUM_CORES + lax.axis_index("c")
    base = wid * PER_W
    pltpu.sync_copy(idx_hbm.at[pl.ds(base, PER_W)], idx_v)

    rows = (rows0, rows1)
    gsem = (gsem0, gsem1)
    ssem = (ssem0, ssem1)

    def g_desc(j, b):
        return pltpu.make_async_copy(
            table_hbm.at[idx_v.at[pl.ds(j * CHUNK, CHUNK)]], rows[b], gsem[b])

    def s_desc(j, b):
        return pltpu.make_async_copy(
            rows[b],
            out_hbm.at[pl.ds(base + j * CHUNK, CHUNK), pl.ds(0, EMBED_DIM)],
            ssem[b])

    g_desc(0, 0).start()
    g_desc(1, 1).start()

    def step(j0, carry):
        for b in range(2):
            j = j0 * 2 + b
            g_desc(j, b).wait()
            s_desc(j, b).start()

            @pl.when(j + 2 < NCHUNK)
            def _():
                s_desc(j, b).wait()
                g_desc(j + 2, b).start()

        return carry

    lax.fori_loop(0, NCHUNK // 2, step, 0, unroll=False)

    s_desc(NCHUNK - 2, 0).wait()
    s_desc(NCHUNK - 1, 1).wait()


@jax.jit
def _run(indices, table):
    table2 = jnp.pad(table, ((0, 0), (0, PAD_DIM - EMBED_DIM)))
    table2 = table2.reshape(2 * VOCAB_ROWS, EMBED_DIM)
    idx2 = indices.reshape(-1) * 2
    mesh = plsc.VectorSubcoreMesh(core_axis_name="c", subcore_axis_name="s")
    grab = pl.kernel(
        _gather_body,
        out_type=jax.ShapeDtypeStruct((TOTAL, PAD_DIM), jnp.float32),
        mesh=mesh,
        scratch_types=[
            pltpu.VMEM((PER_W,), jnp.int32),
            pltpu.VMEM((CHUNK, EMBED_DIM), jnp.float32),
            pltpu.VMEM((CHUNK, EMBED_DIM), jnp.float32),
            pltpu.SemaphoreType.DMA,
            pltpu.SemaphoreType.DMA,
            pltpu.SemaphoreType.DMA,
            pltpu.SemaphoreType.DMA,
        ],
        compiler_params=pltpu.CompilerParams(use_tc_tiling_on_sc=False),
    )
    out = grab(table2, idx2)
    out = out.reshape(indices.shape[0], indices.shape[1], PAD_DIM)
    return out[:, :, :EMBED_DIM]


def kernel(indices, embedding_table):
    return _run(indices, embedding_table)


# speedup vs baseline: 1.4493x; 1.0689x over previous
import functools

import jax
import jax.numpy as jnp
from jax import lax
from jax.experimental import pallas as pl
from jax.experimental.pallas import tpu as pltpu
from jax.experimental.pallas import tpu_sc as plsc

VOCAB_ROWS = 1000000
EMBED_DIM = 64
PAD_DIM = 128
TOTAL = 4096 * 200

_info = plsc.get_sparse_core_info()
NUM_CORES = _info.num_cores
NUM_SUBCORES = _info.num_subcores
NW = NUM_CORES * NUM_SUBCORES
PER_W = TOTAL // NW
CHUNK = 800
NCHUNK = PER_W // CHUNK


def _gather_body(table_hbm, idx_hbm, out_hbm, idx_v,
                 rows0, rows1, gsem0, gsem1, ssem0, ssem1):
    wid = lax.axis_index("s") * NUM_CORES + lax.axis_index("c")
    base = wid * PER_W
    pltpu.sync_copy(idx_hbm.at[pl.ds(base, PER_W)], idx_v)

    rows = (rows0, rows1)
    gsem = (gsem0, gsem1)
    ssem = (ssem0, ssem1)

    def g_desc(j, b):
        return pltpu.make_async_copy(
            table_hbm.at[idx_v.at[pl.ds(j * CHUNK, CHUNK)]], rows[b], gsem[b])

    def s_desc(j, b):
        return pltpu.make_async_copy(
            rows[b],
            out_hbm.at[pl.ds(base + j * CHUNK, CHUNK), pl.ds(0, EMBED_DIM)],
            ssem[b])

    g_desc(0, 0).start()
    g_desc(1, 1).start()

    def step(j0, carry):
        for b in range(2):
            j = j0 * 2 + b
            g_desc(j, b).wait()
            s_desc(j, b).start()

            @pl.when(j + 2 < NCHUNK)
            def _():
                s_desc(j, b).wait()
                g_desc(j + 2, b).start()

        return carry

    lax.fori_loop(0, NCHUNK // 2, step, 0, unroll=False)

    s_desc(NCHUNK - 2, 0).wait()
    s_desc(NCHUNK - 1, 1).wait()


TCB = 2048


def _tc_transpose_body(x_ref, o_ref):
    blk_t = jnp.transpose(x_ref[...])
    o_ref[...] = jnp.concatenate([blk_t, jnp.zeros_like(blk_t)], axis=1)


def _prep_table(table):
    grid = (VOCAB_ROWS + TCB - 1) // TCB
    return pl.pallas_call(
        _tc_transpose_body,
        grid=(grid,),
        in_specs=[pl.BlockSpec((EMBED_DIM, TCB), lambda j: (0, j))],
        out_specs=pl.BlockSpec((TCB, PAD_DIM), lambda j: (j, 0)),
        out_shape=jax.ShapeDtypeStruct((VOCAB_ROWS, PAD_DIM), jnp.float32),
    )(table.T)


@jax.jit
def _run(indices, table):
    table2 = _prep_table(table)
    table2 = table2.reshape(2 * VOCAB_ROWS, EMBED_DIM)
    idx2 = indices.reshape(-1) * 2
    mesh = plsc.VectorSubcoreMesh(core_axis_name="c", subcore_axis_name="s")
    grab = pl.kernel(
        _gather_body,
        out_type=jax.ShapeDtypeStruct((TOTAL, PAD_DIM), jnp.float32),
        mesh=mesh,
        scratch_types=[
            pltpu.VMEM((PER_W,), jnp.int32),
            pltpu.VMEM((CHUNK, EMBED_DIM), jnp.float32),
            pltpu.VMEM((CHUNK, EMBED_DIM), jnp.float32),
            pltpu.SemaphoreType.DMA,
            pltpu.SemaphoreType.DMA,
            pltpu.SemaphoreType.DMA,
            pltpu.SemaphoreType.DMA,
        ],
        compiler_params=pltpu.CompilerParams(use_tc_tiling_on_sc=False),
    )
    out = grab(table2, idx2)
    out = out.reshape(indices.shape[0], indices.shape[1], PAD_DIM)
    return out[:, :, :EMBED_DIM]


def kernel(indices, embedding_table):
    return _run(indices, embedding_table)
